# Initial kernel scaffold; baseline (speedup 1.0000x reference)
import functools

import jax
import jax.numpy as jnp
from jax import lax
from jax.experimental import pallas as pl
from jax.experimental.pallas import tpu as pltpu
from jax.experimental.pallas import tpu_sc as plsc

N_NODES = 10000
N_EDGES = 320000
C = 128

NC = 2
NS = 16
NW = NC * NS

EROWS = N_EDGES // C
ROWS_PER_W = EROWS // NW
EXTRA = EROWS - ROWS_PER_W * NW
NODE_CHUNK = N_NODES // NS


def _sc_aggregate(x, src2d, dst2d, z128, z16, ones16):
  mesh = plsc.VectorSubcoreMesh(core_axis_name="c", subcore_axis_name="s")

  @functools.partial(
      pl.kernel,
      mesh=mesh,
      out_type=[
          jax.ShapeDtypeStruct((NC, N_NODES, C), jnp.float32),
          jax.ShapeDtypeStruct((NC, N_NODES, 16), jnp.float32),
      ],
      scratch_types=[
          pltpu.VMEM((ROWS_PER_W + 1, C), jnp.int32),
          pltpu.VMEM((ROWS_PER_W + 1, C), jnp.int32),
          pltpu.VMEM((C, C), jnp.float32),
          pltpu.VMEM((C, 16), jnp.float32),
          pltpu.VMEM_SHARED((N_NODES, C), jnp.float32),
          pltpu.VMEM_SHARED((N_NODES, 16), jnp.float32),
          pltpu.SemaphoreType.DMA,
      ],
  )
  def k(x_hbm, src_hbm, dst_hbm, z128_hbm, z16_hbm, ones_hbm,
        acc_out, cnt_out, sidx, didx, rows, ones_v, acc_sh, cnt_sh, sem):
    c = lax.axis_index("c")
    s = lax.axis_index("s")
    w = s * NC + c

    nb = s * NODE_CHUNK
    pltpu.sync_copy(z128_hbm.at[pl.ds(nb, NODE_CHUNK)],
                    acc_sh.at[pl.ds(nb, NODE_CHUNK)])
    pltpu.sync_copy(z16_hbm.at[pl.ds(nb, NODE_CHUNK)],
                    cnt_sh.at[pl.ds(nb, NODE_CHUNK)])
    pltpu.sync_copy(ones_hbm, ones_v)

    base = ROWS_PER_W * w + jnp.minimum(w, EXTRA)
    pltpu.sync_copy(src_hbm.at[pl.ds(base, ROWS_PER_W)],
                    sidx.at[pl.ds(0, ROWS_PER_W)])
    pltpu.sync_copy(dst_hbm.at[pl.ds(base, ROWS_PER_W)],
                    didx.at[pl.ds(0, ROWS_PER_W)])

    @pl.when(w < EXTRA)
    def _():
      pltpu.sync_copy(src_hbm.at[base + ROWS_PER_W], sidx.at[ROWS_PER_W])
      pltpu.sync_copy(dst_hbm.at[base + ROWS_PER_W], didx.at[ROWS_PER_W])

    plsc.subcore_barrier()

    def step(i, carry):
      pltpu.async_copy(x_hbm.at[sidx.at[i]], rows, sem).wait()
      pltpu.sync_copy(rows, acc_sh.at[didx.at[i]], add=True)
      pltpu.sync_copy(ones_v, cnt_sh.at[didx.at[i]], add=True)
      return carry

    lax.fori_loop(0, ROWS_PER_W, step, 0)

    @pl.when(w < EXTRA)
    def _():
      step(ROWS_PER_W, 0)

    plsc.subcore_barrier()

    pltpu.sync_copy(acc_sh.at[pl.ds(nb, NODE_CHUNK)],
                    acc_out.at[c, pl.ds(nb, NODE_CHUNK)])
    pltpu.sync_copy(cnt_sh.at[pl.ds(nb, NODE_CHUNK)],
                    cnt_out.at[c, pl.ds(nb, NODE_CHUNK)])

  return k(x, src2d, dst2d, z128, z16, ones16)


def _tc_body(pacc, pcnt, x, wl, wr, b, out):
  acc = pacc[0] + pacc[1]
  cnt = pcnt[0] + pcnt[1]
  mean = acc / jnp.maximum(cnt[:, 0:1], 1.0)
  y = (jnp.dot(mean, wl[...], preferred_element_type=jnp.float32)
       + b[...]
       + jnp.dot(x[...], wr[...], preferred_element_type=jnp.float32))
  out[...] = jnp.maximum(y, 0.0)


def _tc_finish(pacc, pcnt, x, wlT, wrT, b):
  R = 2000
  grid = (N_NODES // R,)
  return pl.pallas_call(
      _tc_body,
      grid=grid,
      in_specs=[
          pl.BlockSpec((NC, R, C), lambda i: (0, i, 0)),
          pl.BlockSpec((NC, R, 16), lambda i: (0, i, 0)),
          pl.BlockSpec((R, C), lambda i: (i, 0)),
          pl.BlockSpec((C, C), lambda i: (0, 0)),
          pl.BlockSpec((C, C), lambda i: (0, 0)),
          pl.BlockSpec((1, C), lambda i: (0, 0)),
      ],
      out_specs=pl.BlockSpec((R, C), lambda i: (i, 0)),
      out_shape=jax.ShapeDtypeStruct((N_NODES, C), jnp.float32),
  )(pacc, pcnt, x, wlT, wrT, b)


def kernel(x, edge_index, W_l, b_l, W_r):
  src2d = edge_index[0].reshape(EROWS, C)
  dst2d = edge_index[1].reshape(EROWS, C)
  z128 = jnp.zeros((N_NODES, C), jnp.float32)
  z16 = jnp.zeros((N_NODES, 16), jnp.float32)
  ones16 = jnp.ones((C, 16), jnp.float32)
  pacc, pcnt = _sc_aggregate(x, src2d, dst2d, z128, z16, ones16)
  return _tc_finish(pacc, pcnt, x, W_l.T, W_r.T, b_l.reshape(1, C))


# speedup vs baseline: 9.7173x; 9.7173x over previous
import functools

import jax
import jax.numpy as jnp
from jax import lax
from jax.experimental import pallas as pl
from jax.experimental.pallas import tpu as pltpu
from jax.experimental.pallas import tpu_sc as plsc

N_NODES = 10000
N_EDGES = 320000
C = 128

NC = 2
NS = 16
NW = NC * NS

ROWS_PER_W = 80
EROWS_PAD = ROWS_PER_W * NW
N_PAD = 10240
NODE_CHUNK = N_PAD // NS

_SC_MESH = dict(core_axis_name="c", subcore_axis_name="s")


def _sc_sum(x, src2d, dst2d):
  @functools.partial(
      pl.kernel,
      mesh=plsc.VectorSubcoreMesh(**_SC_MESH),
      compiler_params=pltpu.CompilerParams(use_tc_tiling_on_sc=False),
      out_type=jax.ShapeDtypeStruct((NC, N_PAD, C), jnp.float32),
      scratch_types=[
          pltpu.VMEM((ROWS_PER_W, C), jnp.int32),
          pltpu.VMEM((ROWS_PER_W, C), jnp.int32),
          pltpu.VMEM((C, C), jnp.float32),
          pltpu.VMEM_SHARED((N_PAD, C), jnp.float32),
          pltpu.SemaphoreType.DMA,
      ],
  )
  def k(x_hbm, src_hbm, dst_hbm, acc_out, sidx, didx, rows, acc_sh, sem):
    c = lax.axis_index("c")
    s = lax.axis_index("s")
    w = s * NC + c

    def fill_row(i, _):
      def fill_lane(j, _):
        rows[i, pl.ds(j * 16, 16)] = jnp.zeros((16,), jnp.float32)
        return 0
      lax.fori_loop(0, C // 16, fill_lane, 0)
      return 0
    lax.fori_loop(0, C, fill_row, 0)

    nb = pl.multiple_of(s * NODE_CHUNK, 8)

    def zero_chunk(j, _):
      pltpu.sync_copy(rows, acc_sh.at[pl.ds(pl.multiple_of(nb + j * C, 8), C)])
      return 0
    lax.fori_loop(0, NODE_CHUNK // C, zero_chunk, 0)

    base = pl.multiple_of(ROWS_PER_W * w, 8)
    pltpu.sync_copy(src_hbm.at[pl.ds(base, ROWS_PER_W)], sidx)
    pltpu.sync_copy(dst_hbm.at[pl.ds(base, ROWS_PER_W)], didx)

    plsc.subcore_barrier()

    def step(i, carry):
      pltpu.async_copy(x_hbm.at[sidx.at[i]], rows, sem).wait()
      pltpu.sync_copy(rows, acc_sh.at[didx.at[i]], add=True)
      return carry

    lax.fori_loop(0, ROWS_PER_W, step, 0)

    plsc.subcore_barrier()

    pltpu.sync_copy(acc_sh.at[pl.ds(nb, NODE_CHUNK)],
                    acc_out.at[c, pl.ds(nb, NODE_CHUNK)])

  return k(x, src2d, dst2d)


def _sc_count(dst2d):
  @functools.partial(
      pl.kernel,
      mesh=plsc.VectorSubcoreMesh(**_SC_MESH),
      compiler_params=pltpu.CompilerParams(use_tc_tiling_on_sc=False),
      out_type=jax.ShapeDtypeStruct((NC, N_PAD, 16), jnp.float32),
      scratch_types=[
          pltpu.VMEM((ROWS_PER_W, C), jnp.int32),
          pltpu.VMEM((C, 16), jnp.float32),
          pltpu.VMEM((C, 16), jnp.float32),
          pltpu.VMEM_SHARED((N_PAD, 16), jnp.float32),
      ],
  )
  def k(dst_hbm, cnt_out, didx, ones_v, zc_v, cnt_sh):
    c = lax.axis_index("c")
    s = lax.axis_index("s")
    w = s * NC + c

    def fill_row(i, _):
      ones_v[i] = jnp.ones((16,), jnp.float32)
      zc_v[i] = jnp.zeros((16,), jnp.float32)
      return 0
    lax.fori_loop(0, C, fill_row, 0)

    nb = pl.multiple_of(s * NODE_CHUNK, 8)

    def zero_chunk(j, _):
      pltpu.sync_copy(zc_v, cnt_sh.at[pl.ds(pl.multiple_of(nb + j * C, 8), C)])
      return 0
    lax.fori_loop(0, NODE_CHUNK // C, zero_chunk, 0)

    base = pl.multiple_of(ROWS_PER_W * w, 8)
    pltpu.sync_copy(dst_hbm.at[pl.ds(base, ROWS_PER_W)], didx)

    plsc.subcore_barrier()

    def step(i, carry):
      pltpu.sync_copy(ones_v, cnt_sh.at[didx.at[i]], add=True)
      return carry

    lax.fori_loop(0, ROWS_PER_W, step, 0)

    plsc.subcore_barrier()

    pltpu.sync_copy(cnt_sh.at[pl.ds(nb, NODE_CHUNK)],
                    cnt_out.at[c, pl.ds(nb, NODE_CHUNK)])

  return k(dst2d)


def _tc_body(pacc, pcnt, x, wl, wr, b, out):
  acc = pacc[0] + pacc[1]
  cnt = pcnt[0] + pcnt[1]
  mean = acc / jnp.maximum(cnt[:, 0:1], 1.0)
  y = (jnp.dot(mean, wl[...], preferred_element_type=jnp.float32)
       + b[...]
       + jnp.dot(x[...], wr[...], preferred_element_type=jnp.float32))
  out[...] = jnp.maximum(y, 0.0)


def _tc_finish(pacc, pcnt, x, wlT, wrT, b):
  R = 2000
  grid = (N_NODES // R,)
  return pl.pallas_call(
      _tc_body,
      grid=grid,
      in_specs=[
          pl.BlockSpec((NC, R, C), lambda i: (0, i, 0)),
          pl.BlockSpec((NC, R, 16), lambda i: (0, i, 0)),
          pl.BlockSpec((R, C), lambda i: (i, 0)),
          pl.BlockSpec((C, C), lambda i: (0, 0)),
          pl.BlockSpec((C, C), lambda i: (0, 0)),
          pl.BlockSpec((1, C), lambda i: (0, 0)),
      ],
      out_specs=pl.BlockSpec((R, C), lambda i: (i, 0)),
      out_shape=jax.ShapeDtypeStruct((N_NODES, C), jnp.float32),
  )(pacc, pcnt, x, wlT, wrT, b)


_N_EDGE_PAD = EROWS_PAD * C - N_EDGES


def kernel(x, edge_index, W_l, b_l, W_r):
  ar = jnp.arange(_N_EDGE_PAD, dtype=jnp.int32)
  pad_src = ar % N_NODES
  pad_dst = ar % (N_PAD - N_NODES) + N_NODES
  src2d = jnp.concatenate([edge_index[0], pad_src]).reshape(EROWS_PAD, C)
  dst2d = jnp.concatenate([edge_index[1], pad_dst]).reshape(EROWS_PAD, C)
  pacc = _sc_sum(x, src2d, dst2d)
  pcnt = _sc_count(dst2d)
  return _tc_finish(pacc, pcnt, x, W_l.T, W_r.T, b_l.reshape(1, C))


# speedup vs baseline: 12.1423x; 1.2496x over previous
import functools

import jax
import jax.numpy as jnp
from jax import lax
from jax.experimental import pallas as pl
from jax.experimental.pallas import tpu as pltpu
from jax.experimental.pallas import tpu_sc as plsc

N_NODES = 10000
N_EDGES = 320000
C = 128

NC = 2
NS = 16
NW = NC * NS

ROWS_PER_W = 80
EROWS_PAD = ROWS_PER_W * NW
W = 64
CHUNKS_PER_W = ROWS_PER_W * C // W
N_PAD = 10240
NODE_CHUNK = N_PAD // NS

_SC_MESH = dict(core_axis_name="c", subcore_axis_name="s")


def _sc_sum(x, sd_slab):
  @functools.partial(
      pl.kernel,
      mesh=plsc.VectorSubcoreMesh(**_SC_MESH),
      compiler_params=pltpu.CompilerParams(use_tc_tiling_on_sc=False),
      out_type=jax.ShapeDtypeStruct((NC, N_PAD, C), jnp.float32),
      scratch_types=[
          pltpu.VMEM((2 * CHUNKS_PER_W, W), jnp.int32),
          pltpu.VMEM((2, W, C), jnp.float32),
          pltpu.VMEM_SHARED((N_PAD, C), jnp.float32),
          pltpu.SemaphoreType.DMA((2,)),
      ],
  )
  def k(x_hbm, sd_hbm, acc_out, slab, rows2, acc_sh, sem2):
    c = lax.axis_index("c")
    s = lax.axis_index("s")
    w = s * NC + c

    def fill_row(i, _):
      def fill_lane(j, _):
        rows2[0, i, pl.ds(j * 16, 16)] = jnp.zeros((16,), jnp.float32)
        return 0
      lax.fori_loop(0, C // 16, fill_lane, 0)
      return 0
    lax.fori_loop(0, W, fill_row, 0)

    nb = pl.multiple_of(s * NODE_CHUNK, 8)

    def zero_chunk(j, _):
      pltpu.sync_copy(rows2.at[0],
                      acc_sh.at[pl.ds(pl.multiple_of(nb + j * W, 8), W)])
      return 0
    lax.fori_loop(0, NODE_CHUNK // W, zero_chunk, 0)

    base = pl.multiple_of(2 * CHUNKS_PER_W * w, 8)
    pltpu.sync_copy(sd_hbm.at[pl.ds(base, 2 * CHUNKS_PER_W)], slab)

    plsc.subcore_barrier()

    def step(i, carry):
      b = jnp.bitwise_and(i, 1)

      @pl.when(i < CHUNKS_PER_W)
      def _():
        pltpu.async_copy(x_hbm.at[slab.at[i]], rows2.at[b], sem2.at[b])

      @pl.when(i > 0)
      def _():
        pb = jnp.bitwise_and(i - 1, 1)
        pltpu.make_async_copy(
            x_hbm.at[pl.ds(0, W)], rows2.at[pb], sem2.at[pb]).wait()
        pltpu.sync_copy(rows2.at[pb],
                        acc_sh.at[slab.at[CHUNKS_PER_W + i - 1]], add=True)
      return carry

    lax.fori_loop(0, CHUNKS_PER_W + 1, step, 0)

    plsc.subcore_barrier()

    def wb(j, _):
      off = pl.multiple_of(nb + j * C, 8)
      pltpu.sync_copy(acc_sh.at[pl.ds(off, C)], acc_out.at[c, pl.ds(off, C)])
      return 0
    lax.fori_loop(0, NODE_CHUNK // C, wb, 0)

  return k(x, sd_slab)


def _sc_count(dst2d):
  @functools.partial(
      pl.kernel,
      mesh=plsc.VectorSubcoreMesh(**_SC_MESH),
      compiler_params=pltpu.CompilerParams(use_tc_tiling_on_sc=False),
      out_type=jax.ShapeDtypeStruct((NC, N_PAD, 16), jnp.float32),
      scratch_types=[
          pltpu.VMEM((ROWS_PER_W, C), jnp.int32),
          pltpu.VMEM((C, 16), jnp.float32),
          pltpu.VMEM((C, 16), jnp.float32),
          pltpu.VMEM_SHARED((N_PAD, 16), jnp.float32),
      ],
  )
  def k(dst_hbm, cnt_out, didx, ones_v, zc_v, cnt_sh):
    c = lax.axis_index("c")
    s = lax.axis_index("s")
    w = s * NC + c

    def fill_row(i, _):
      ones_v[i] = jnp.ones((16,), jnp.float32)
      zc_v[i] = jnp.zeros((16,), jnp.float32)
      return 0
    lax.fori_loop(0, C, fill_row, 0)

    nb = pl.multiple_of(s * NODE_CHUNK, 8)

    def zero_chunk(j, _):
      pltpu.sync_copy(zc_v, cnt_sh.at[pl.ds(pl.multiple_of(nb + j * C, 8), C)])
      return 0
    lax.fori_loop(0, NODE_CHUNK // C, zero_chunk, 0)

    base = pl.multiple_of(ROWS_PER_W * w, 8)
    pltpu.sync_copy(dst_hbm.at[pl.ds(base, ROWS_PER_W)], didx)

    plsc.subcore_barrier()

    def step(i, carry):
      pltpu.sync_copy(ones_v, cnt_sh.at[didx.at[i]], add=True)
      return carry

    lax.fori_loop(0, ROWS_PER_W, step, 0)

    plsc.subcore_barrier()

    pltpu.sync_copy(cnt_sh.at[pl.ds(nb, NODE_CHUNK)],
                    cnt_out.at[c, pl.ds(nb, NODE_CHUNK)])

  return k(dst2d)


def _tc_body(pacc, pcnt, x, wl, wr, b, out):
  acc = pacc[0] + pacc[1]
  cnt = pcnt[0] + pcnt[1]
  mean = acc / jnp.maximum(cnt[:, 0:1], 1.0)
  y = (jnp.dot(mean, wl[...], preferred_element_type=jnp.float32)
       + b[...]
       + jnp.dot(x[...], wr[...], preferred_element_type=jnp.float32))
  out[...] = jnp.maximum(y, 0.0)


def _tc_finish(pacc, pcnt, x, wlT, wrT, b):
  R = 2000
  grid = (N_NODES // R,)
  return pl.pallas_call(
      _tc_body,
      grid=grid,
      in_specs=[
          pl.BlockSpec((NC, R, C), lambda i: (0, i, 0)),
          pl.BlockSpec((NC, R, 16), lambda i: (0, i, 0)),
          pl.BlockSpec((R, C), lambda i: (i, 0)),
          pl.BlockSpec((C, C), lambda i: (0, 0)),
          pl.BlockSpec((C, C), lambda i: (0, 0)),
          pl.BlockSpec((1, C), lambda i: (0, 0)),
      ],
      out_specs=pl.BlockSpec((R, C), lambda i: (i, 0)),
      out_shape=jax.ShapeDtypeStruct((N_NODES, C), jnp.float32),
  )(pacc, pcnt, x, wlT, wrT, b)


_N_EDGE_PAD = EROWS_PAD * C - N_EDGES


def kernel(x, edge_index, W_l, b_l, W_r):
  ar = jnp.arange(_N_EDGE_PAD, dtype=jnp.int32)
  pad_src = ar % N_NODES
  pad_dst = ar % (N_PAD - N_NODES) + N_NODES
  src3d = jnp.concatenate([edge_index[0], pad_src]).reshape(NW, CHUNKS_PER_W, W)
  dst3d = jnp.concatenate([edge_index[1], pad_dst]).reshape(NW, CHUNKS_PER_W, W)
  sd_slab = jnp.concatenate([src3d, dst3d], axis=1)
  sd_slab = sd_slab.reshape(2 * NW * CHUNKS_PER_W, W)
  pacc = _sc_sum(x, sd_slab)
  pcnt = _sc_count(dst3d.reshape(EROWS_PAD, C))
  return _tc_finish(pacc, pcnt, x, W_l.T, W_r.T, b_l.reshape(1, C))
